# baseline (speedup 1.0000x reference)
import functools

import jax
import jax.numpy as jnp
from jax import lax
from jax.experimental import pallas as pl
from jax.experimental.pallas import tpu as pltpu
from jax.experimental.pallas import tpu_sc as plsc

RANKS = 13
SUITS = 4
RANK_DIM = 8
SUIT_DIM = 4
OUT_DIM = 16
BATCH = 16384

NC = 1
NS = 16
LANES = 16
NW = NC * NS
BPW = BATCH // NW
CHUNK = 128
NCHUNK = BPW // CHUNK


def _fold_body(rank_ref, suit_ref, w_ref, b_ref, out_ref):
    tr = jnp.dot(rank_ref[...], w_ref[:RANK_DIM, :],
                 preferred_element_type=jnp.float32) + b_ref[...]
    ts = jnp.dot(suit_ref[...], w_ref[RANK_DIM:, :],
                 preferred_element_type=jnp.float32)
    out_ref[...] = tr[:, None, :] + ts[None, :, :]


def _fold_tables(rank_emb, suit_emb, W, b):
    t3 = pl.pallas_call(
        _fold_body,
        out_shape=jax.ShapeDtypeStruct((RANKS, SUITS, OUT_DIM), jnp.float32),
    )(rank_emb, suit_emb, W, b.reshape(1, OUT_DIM))
    return t3.reshape(RANKS * SUITS, OUT_DIM)


def _sc_body(r_hbm, s_hbm, table_hbm, out_hbm, r_v, s_v, table_v, rows_v, sem):
    wid = lax.axis_index("s") * NC + lax.axis_index("c")
    base = pl.multiple_of(wid * BPW, 128)
    rcopy = pltpu.async_copy(r_hbm.at[pl.ds(base, BPW)], r_v, sem)
    scopy = pltpu.async_copy(s_hbm.at[pl.ds(base, BPW)], s_v, sem)
    pltpu.sync_copy(table_hbm, table_v)
    rcopy.wait()
    scopy.wait()
    lane = lax.iota(jnp.int32, LANES)
    hi0 = lane >> 3
    lo0 = (lane & 7) * OUT_DIM

    @plsc.parallel_loop(0, BPW // LANES, unroll=8)
    def _chunk(j):
        off = j * LANES
        rv = r_v[pl.ds(off, LANES)]
        sv = s_v[pl.ds(off, LANES)]
        tbase = (rv * SUITS + sv) * OUT_DIM
        hi = hi0 + 2 * j
        for c in range(OUT_DIM):
            vals = plsc.load_gather(table_v, [tbase + c])
            plsc.store_scatter(rows_v, [hi, lo0 + c], vals)

    obase = pl.multiple_of(wid * (BPW * OUT_DIM // 128), 8)
    pltpu.sync_copy(rows_v, out_hbm.at[pl.ds(obase, BPW * OUT_DIM // 128)])


@functools.lru_cache(maxsize=1)
def _sc_lookup():
    return pl.kernel(
        _sc_body,
        mesh=plsc.VectorSubcoreMesh(core_axis_name="c", subcore_axis_name="s",
                                    num_cores=NC),
        out_type=jax.ShapeDtypeStruct((BATCH * OUT_DIM // 128, 128), jnp.float32),
        scratch_types=[
            pltpu.VMEM((BPW,), jnp.int32),
            pltpu.VMEM((BPW,), jnp.int32),
            pltpu.VMEM((RANKS * SUITS * OUT_DIM,), jnp.float32),
            pltpu.VMEM((BPW * OUT_DIM // 128, 128), jnp.float32),
            pltpu.SemaphoreType.DMA,
        ],
        compiler_params=pltpu.CompilerParams(
            needs_layout_passes=False, use_tc_tiling_on_sc=True),
    )


def kernel(card_tensor, rank_emb, suit_emb, W, b):
    table = _fold_tables(rank_emb, suit_emb, W, b)
    r_idx = card_tensor[:, 0].astype(jnp.int32)
    s_idx = card_tensor[:, 1].astype(jnp.int32)
    out128 = _sc_lookup()(r_idx, s_idx, table.reshape(-1))
    return out128.reshape(BATCH, OUT_DIM)


# speedup vs baseline: 1.0464x; 1.0464x over previous
import functools

import jax
import jax.numpy as jnp
from jax import lax
from jax.experimental import pallas as pl
from jax.experimental.pallas import tpu as pltpu
from jax.experimental.pallas import tpu_sc as plsc

RANKS = 13
SUITS = 4
RANK_DIM = 8
SUIT_DIM = 4
OUT_DIM = 16
BATCH = 16384

NC = 1
NS = 16
LANES = 16
NW = NC * NS
BPW = BATCH // NW
CHUNK = 128
NCHUNK = BPW // CHUNK


def _fold_body(rank_ref, suit_ref, w_ref, b_ref, out_ref):
    tr = jnp.dot(rank_ref[...], w_ref[:RANK_DIM, :],
                 preferred_element_type=jnp.float32) + b_ref[...]
    ts = jnp.dot(suit_ref[...], w_ref[RANK_DIM:, :],
                 preferred_element_type=jnp.float32)
    out_ref[...] = tr[:, None, :] + ts[None, :, :]


def _fold_tables(rank_emb, suit_emb, W, b):
    t3 = pl.pallas_call(
        _fold_body,
        out_shape=jax.ShapeDtypeStruct((RANKS, SUITS, OUT_DIM), jnp.float32),
    )(rank_emb, suit_emb, W, b.reshape(1, OUT_DIM))
    return t3.reshape(RANKS * SUITS, OUT_DIM)


def _sc_body(r_hbm, s_hbm, table_hbm, out_hbm, r_v, s_v, table_v, rows_v, sem):
    wid = lax.axis_index("s") * NC + lax.axis_index("c")
    base = pl.multiple_of(wid * BPW, 128)
    rcopy = pltpu.async_copy(r_hbm.at[pl.ds(base, BPW)], r_v, sem)
    scopy = pltpu.async_copy(s_hbm.at[pl.ds(base, BPW)], s_v, sem)
    pltpu.sync_copy(table_hbm, table_v)
    rcopy.wait()
    scopy.wait()
    lane = lax.iota(jnp.int32, LANES)
    hi0 = lane >> 3
    lo0 = (lane & 7) * OUT_DIM

    @plsc.parallel_loop(0, BPW // LANES, unroll=4)
    def _chunk(j):
        off = j * LANES
        rv = r_v[pl.ds(off, LANES)]
        sv = s_v[pl.ds(off, LANES)]
        tbase = (rv * SUITS + sv) * OUT_DIM
        hi = hi0 + 2 * j
        for c in range(OUT_DIM):
            vals = plsc.load_gather(table_v, [tbase + c])
            plsc.store_scatter(rows_v, [hi, lo0 + c], vals)

    obase = pl.multiple_of(wid * (BPW * OUT_DIM // 128), 8)
    pltpu.sync_copy(rows_v, out_hbm.at[pl.ds(obase, BPW * OUT_DIM // 128)])


@functools.lru_cache(maxsize=1)
def _sc_lookup():
    return pl.kernel(
        _sc_body,
        mesh=plsc.VectorSubcoreMesh(core_axis_name="c", subcore_axis_name="s",
                                    num_cores=NC),
        out_type=jax.ShapeDtypeStruct((BATCH * OUT_DIM // 128, 128), jnp.float32),
        scratch_types=[
            pltpu.VMEM((BPW,), jnp.int32),
            pltpu.VMEM((BPW,), jnp.int32),
            pltpu.VMEM((RANKS * SUITS * OUT_DIM,), jnp.float32),
            pltpu.VMEM((BPW * OUT_DIM // 128, 128), jnp.float32),
            pltpu.SemaphoreType.DMA,
        ],
        compiler_params=pltpu.CompilerParams(
            needs_layout_passes=False, use_tc_tiling_on_sc=True),
    )


def kernel(card_tensor, rank_emb, suit_emb, W, b):
    table = _fold_tables(rank_emb, suit_emb, W, b)
    r_idx = card_tensor[:, 0].astype(jnp.int32)
    s_idx = card_tensor[:, 1].astype(jnp.int32)
    out128 = _sc_lookup()(r_idx, s_idx, table.reshape(-1))
    return out128.reshape(BATCH, OUT_DIM)
